# Initial kernel scaffold; baseline (speedup 1.0000x reference)
import jax
import jax.numpy as jnp
from jax.experimental import pallas as pl
from jax.experimental.pallas import tpu as pltpu
from jax.experimental.pallas import tpu_sc as plsc

K = 8192
D = 32
TN = 256


def _score_argmax_body(x_ref, emb_ref, ind_ref):
    x = x_ref[...]
    e = emb_ref[...]
    mm = jax.lax.dot_general(
        x, e, dimension_numbers=(((1,), (1,)), ((), ())),
        preferred_element_type=jnp.float32)
    x2 = jnp.sum(x * x, axis=1, keepdims=True)
    e2 = jnp.sum(e * e, axis=1)[None, :]
    dist = -(x2 - 2.0 * mm + e2)
    m = jnp.max(dist, axis=1, keepdims=True)
    iota = jax.lax.broadcasted_iota(jnp.int32, (TN, K), 1)
    ind = jnp.min(jnp.where(dist == m, iota, K), axis=1)
    ind_ref[0, 0, :] = ind


def _nearest_code(flat, embed):
    n = flat.shape[0]
    g = n // TN
    ind = pl.pallas_call(
        _score_argmax_body,
        grid=(g,),
        in_specs=[
            pl.BlockSpec((TN, D), lambda i: (i, 0)),
            pl.BlockSpec((K, D), lambda i: (0, 0)),
        ],
        out_specs=pl.BlockSpec((1, 1, TN), lambda i: (i, 0, 0)),
        out_shape=jax.ShapeDtypeStruct((g, 1, TN), jnp.int32),
    )(flat, embed)
    return ind.reshape(n)


def _sc_gather(embed, idx_flat):
    n = idx_flat.shape[0]
    w = 128
    mesh = plsc.VectorSubcoreMesh(
        core_axis_name="core", subcore_axis_name="subcore")

    @pl.kernel(out_type=jax.ShapeDtypeStruct((n, D), embed.dtype), mesh=mesh)
    def gather_kernel(emb_hbm, i_hbm, o_hbm):
        def body(i_vmem, o_vmem):
            pltpu.sync_copy(emb_hbm.at[i_vmem.at[0]], o_vmem)

        pltpu.emit_pipeline(
            body,
            grid=(n // w,),
            in_specs=[pl.BlockSpec((1, w), index_map=lambda i: (0, i))],
            out_specs=[pl.BlockSpec((w, D), index_map=lambda i: (i, 0))],
            core_axis_name=("core", "subcore"),
            dimension_semantics=(pltpu.PARALLEL,),
        )(i_hbm, o_hbm)

    return gather_kernel(embed, idx_flat.reshape(1, n))


def kernel(x, inited, cluster_size, embed, embed_avg):
    shape = x.shape
    flat = x.reshape(-1, shape[-1])
    ind_flat = _nearest_code(flat, embed)
    quantize = _sc_gather(embed, ind_flat).reshape(shape)
    embed_ind = ind_flat.reshape(shape[:-1])
    return (quantize, embed_ind)


# speedup vs baseline: 1.0480x; 1.0480x over previous
import functools

import jax
import jax.numpy as jnp
from jax.experimental import pallas as pl
from jax.experimental.pallas import tpu as pltpu
from jax.experimental.pallas import tpu_sc as plsc

K = 8192
D = 32
TN = 256


def _score_argmax_body(x_ref, emb_ref, x2_ref, e2_ref, ind_ref):
    x = x_ref[...]
    e = emb_ref[...]
    mm = jax.lax.dot_general(
        x.astype(jnp.bfloat16), e.astype(jnp.bfloat16),
        dimension_numbers=(((1,), (1,)), ((), ())),
        preferred_element_type=jnp.float32)
    dist = -(x2_ref[...] - 2.0 * mm + e2_ref[...])
    m = jnp.max(dist, axis=1, keepdims=True)
    iota = jax.lax.broadcasted_iota(jnp.int32, (TN, K), 1)
    ind = jnp.min(jnp.where(dist == m, iota, K), axis=1)
    ind_ref[0, 0, :] = ind


def _nearest_code(flat, embed):
    n = flat.shape[0]
    g = n // TN
    et = embed.T
    x2 = jnp.sum(flat * flat, axis=1, keepdims=True)
    e2 = jnp.sum(et * et, axis=0, keepdims=True)
    ind = pl.pallas_call(
        _score_argmax_body,
        grid=(g,),
        in_specs=[
            pl.BlockSpec((TN, D), lambda i: (i, 0)),
            pl.BlockSpec((K, D), lambda i: (0, 0)),
            pl.BlockSpec((TN, 1), lambda i: (i, 0)),
            pl.BlockSpec((1, K), lambda i: (0, 0)),
        ],
        out_specs=pl.BlockSpec((1, 1, TN), lambda i: (i, 0, 0)),
        out_shape=jax.ShapeDtypeStruct((g, 1, TN), jnp.int32),
    )(flat, embed, x2, e2)
    return ind.reshape(n)


def _sc_gather(embed, idx_flat):
    n = idx_flat.shape[0]
    k = embed.shape[0]
    w = 128
    table = jnp.pad(embed, ((0, 0), (0, w - D)))
    nc, ns = 2, 16
    nw = nc * ns
    b_per_w = n // nw
    mesh = plsc.VectorSubcoreMesh(core_axis_name="c", subcore_axis_name="s")

    @functools.partial(
        pl.kernel, mesh=mesh,
        out_type=jax.ShapeDtypeStruct((n, w), jnp.float32),
        scratch_types=[
            pltpu.VMEM((b_per_w,), jnp.int32),
            pltpu.VMEM((b_per_w, w), jnp.float32),
            pltpu.SemaphoreType.DMA,
        ],
    )
    def gather_kernel(table_hbm, idx_hbm, out_hbm, idx_v, rows_v, sem):
        wid = jax.lax.axis_index("s") * nc + jax.lax.axis_index("c")
        base = wid * b_per_w
        pltpu.sync_copy(idx_hbm.at[pl.ds(base, b_per_w)], idx_v)
        pltpu.async_copy(table_hbm.at[idx_v], rows_v, sem).wait()
        pltpu.sync_copy(rows_v, out_hbm.at[pl.ds(base, b_per_w)])

    return gather_kernel(table, idx_flat)[:, :D]


def kernel(x, inited, cluster_size, embed, embed_avg):
    shape = x.shape
    flat = x.reshape(-1, shape[-1])
    ind_flat = _nearest_code(flat, embed)
    quantize = _sc_gather(embed, ind_flat).reshape(shape)
    embed_ind = ind_flat.reshape(shape[:-1])
    return (quantize, embed_ind)
